# baseline (speedup 1.0000x reference)
import functools

import jax
import jax.numpy as jnp
from jax import lax
from jax.experimental import pallas as pl
from jax.experimental.pallas import tpu as pltpu
from jax.experimental.pallas import tpu_sc as plsc

_D = 32
_NC = 2
_NS = 16
_NW = _NC * _NS
_B = 4096
_L = 200
_TR = _L // 8
_BB = _B // 128
_EB = _D // 8


@functools.cache
def _make_emb():
    mesh = plsc.VectorSubcoreMesh(core_axis_name="c", subcore_axis_name="s")

    @functools.partial(
        pl.kernel,
        mesh=mesh,
        out_type=jax.ShapeDtypeStruct((_L, _EB, _BB, 8, 128), jnp.float32),
        scratch_types=[
            pltpu.VMEM((_TR, 8, 128), jnp.int32),
            pltpu.VMEM((2, 128, _D), jnp.float32),
            pltpu.VMEM((2, _D, 128), jnp.float32),
            pltpu.SemaphoreType.DMA,
            pltpu.SemaphoreType.DMA,
            pltpu.SemaphoreType.DMA,
            pltpu.SemaphoreType.DMA,
        ],
        compiler_params=pltpu.CompilerParams(
            use_tc_tiling_on_sc=False, needs_layout_passes=False),
    )
    def emb(idx_hbm, table_hbm, out_hbm, idx_v, rows_v, tile_v, g0, g1, w0, w1):
        w = lax.axis_index("s") * _NC + lax.axis_index("c")
        pltpu.sync_copy(idx_hbm.at[:, w], idx_v)

        def issue_gather(l, b, sem):
            return pltpu.async_copy(
                table_hbm.at[idx_v.at[l // 8, l % 8]], rows_v.at[b], sem)

        def wait_gather(b, sem):
            pltpu.make_async_copy(
                table_hbm.at[idx_v.at[0, 0]], rows_v.at[b], sem).wait()

        def issue_writes(l, b, sem):
            for eb in range(_EB):
                pltpu.async_copy(
                    tile_v.at[b, pl.ds(eb * 8, 8)], out_hbm.at[l, eb, w], sem)

        def wait_writes(b, sem):
            for eb in range(_EB):
                pltpu.make_async_copy(
                    tile_v.at[b, pl.ds(eb * 8, 8)], out_hbm.at[0, eb, 0], sem
                ).wait()

        iotas = [lax.iota(jnp.int32, 16) + blk * 16 for blk in range(8)]

        def transpose(b):
            @plsc.parallel_loop(0, _D, 1, unroll=8)
            def tbody(e):
                col = jnp.full((16,), e, jnp.int32)
                for blk in range(8):
                    v = plsc.load_gather(rows_v.at[b], [iotas[blk], col])
                    tile_v[b, e, pl.ds(blk * 16, 16)] = v

        issue_gather(0, 0, g0)
        issue_gather(1, 1, g1)

        def body(p, carry):
            for half, (b, gs, ws) in enumerate(((0, g0, w0), (1, g1, w1))):
                l = 2 * p + half
                wait_gather(b, gs)

                @pl.when(l >= 2)
                def _():
                    wait_writes(b, ws)

                transpose(b)
                issue_writes(l, b, ws)

                @pl.when(l + 2 < _L)
                def _():
                    issue_gather(l + 2, b, gs)
            return carry

        lax.fori_loop(0, _L // 2, body, 0)
        wait_writes(0, w0)
        wait_writes(1, w1)

    return emb


def kernel(x, table):
    idx = x.reshape(_BB, 128, _TR, 8).transpose(2, 0, 3, 1)
    out = _make_emb()(idx, table)
    return out.transpose(2, 4, 0, 1, 3).reshape(_B, _L, _D)


# speedup vs baseline: 1.0262x; 1.0262x over previous
import functools

import jax
import jax.numpy as jnp
from jax import lax
from jax.experimental import pallas as pl
from jax.experimental.pallas import tpu as pltpu
from jax.experimental.pallas import tpu_sc as plsc

_D = 32
_NC = 2
_NS = 16
_NW = _NC * _NS
_B = 4096
_L = 200
_TR = _L // 8
_BB = _B // 128
_EB = _D // 8
_NBUF = 8


@functools.cache
def _make_emb():
    mesh = plsc.VectorSubcoreMesh(core_axis_name="c", subcore_axis_name="s")

    @functools.partial(
        pl.kernel,
        mesh=mesh,
        out_type=jax.ShapeDtypeStruct((_L, _EB, _BB, 8, 128), jnp.float32),
        scratch_types=[
            pltpu.VMEM((_TR, 8, 128), jnp.int32),
            pltpu.VMEM((_NBUF, 128, _D), jnp.float32),
            pltpu.VMEM((_NBUF, _D, 128), jnp.float32),
            pltpu.SemaphoreType.DMA((_NBUF,)),
            pltpu.SemaphoreType.DMA((_NBUF,)),
        ],
        compiler_params=pltpu.CompilerParams(
            use_tc_tiling_on_sc=False, needs_layout_passes=False),
    )
    def emb(idx_hbm, table_hbm, out_hbm, idx_v, rows_v, tile_v, gsem, wsem):
        w = lax.axis_index("s") * _NC + lax.axis_index("c")
        pltpu.sync_copy(idx_hbm.at[:, w], idx_v)

        def issue_gather(l, b):
            return pltpu.async_copy(
                table_hbm.at[idx_v.at[l // 8, l % 8]], rows_v.at[b],
                gsem.at[b])

        def wait_gather(b):
            pltpu.make_async_copy(
                table_hbm.at[idx_v.at[0, 0]], rows_v.at[b], gsem.at[b]).wait()

        def issue_writes(l, b):
            for eb in range(_EB):
                pltpu.async_copy(
                    tile_v.at[b, pl.ds(eb * 8, 8)], out_hbm.at[l, eb, w],
                    wsem.at[b])

        def wait_writes(b):
            for eb in range(_EB):
                pltpu.make_async_copy(
                    tile_v.at[b, pl.ds(eb * 8, 8)], out_hbm.at[0, eb, 0],
                    wsem.at[b]).wait()

        iotas = [lax.iota(jnp.int32, 16) + blk * 16 for blk in range(8)]

        def transpose(b):
            @plsc.parallel_loop(0, _D, 1, unroll=2)
            def tbody(e):
                col = jnp.full((16,), e, jnp.int32)
                for blk in range(8):
                    v = plsc.load_gather(rows_v.at[b], [iotas[blk], col])
                    tile_v[b, e, pl.ds(blk * 16, 16)] = v

        for b in range(_NBUF):
            issue_gather(b, b)

        def body(g, carry):
            for b in range(_NBUF):
                l = _NBUF * g + b
                wait_gather(b)

                @pl.when(g > 0)
                def _():
                    wait_writes(b)

                transpose(b)
                issue_writes(l, b)

                @pl.when(l + _NBUF < _L)
                def _():
                    issue_gather(l + _NBUF, b)
            return carry

        lax.fori_loop(0, _L // _NBUF, body, 0)
        for b in range(_NBUF):
            wait_writes(b)

    return emb


def kernel(x, table):
    idx = x.reshape(_BB, 128, _TR, 8).transpose(2, 0, 3, 1)
    out = _make_emb()(idx, table)
    return out.transpose(2, 4, 0, 1, 3).reshape(_B, _L, _D)


# speedup vs baseline: 1.4625x; 1.4251x over previous
import functools

import jax
import jax.numpy as jnp
from jax import lax
from jax.experimental import pallas as pl
from jax.experimental.pallas import tpu as pltpu
from jax.experimental.pallas import tpu_sc as plsc

_D = 32
_NC = 2
_NS = 16
_NW = _NC * _NS
_B = 4096
_L = 200
_TR = _L // 8
_BB = _B // 128
_EB = _D // 8
_NBUF = 4
_P = _D + 1


@functools.cache
def _make_emb():
    mesh = plsc.VectorSubcoreMesh(core_axis_name="c", subcore_axis_name="s")

    @functools.partial(
        pl.kernel,
        mesh=mesh,
        out_type=jax.ShapeDtypeStruct((_L, _EB, _BB, 8, 128), jnp.float32),
        scratch_types=[
            pltpu.VMEM((_TR, 8, 128), jnp.int32),
            pltpu.VMEM((_NBUF, 128, _D), jnp.float32),
            pltpu.VMEM((2, 128, _P), jnp.float32),
            pltpu.VMEM((2, _D, 128), jnp.float32),
            pltpu.SemaphoreType.DMA((_NBUF,)),
            pltpu.SemaphoreType.DMA((2,)),
        ],
        compiler_params=pltpu.CompilerParams(
            use_tc_tiling_on_sc=False, needs_layout_passes=False),
    )
    def emb(idx_hbm, table_hbm, out_hbm, idx_v, rows_v, skew_v, tile_v,
            gsem, wsem):
        w = lax.axis_index("s") * _NC + lax.axis_index("c")
        pltpu.sync_copy(idx_hbm.at[:, w], idx_v)

        def issue_gather(l, b):
            return pltpu.async_copy(
                table_hbm.at[idx_v.at[l // 8, l % 8]], rows_v.at[b],
                gsem.at[b])

        def wait_gather(b):
            pltpu.make_async_copy(
                table_hbm.at[idx_v.at[0, 0]], rows_v.at[b], gsem.at[b]).wait()

        def issue_writes(l, c):
            for eb in range(_EB):
                pltpu.async_copy(
                    tile_v.at[c, pl.ds(eb * 8, 8)], out_hbm.at[l, eb, w],
                    wsem.at[c])

        def wait_writes(c):
            for eb in range(_EB):
                pltpu.make_async_copy(
                    tile_v.at[c, pl.ds(eb * 8, 8)], out_hbm.at[0, eb, 0],
                    wsem.at[c]).wait()

        def repack(b, c):
            @plsc.parallel_loop(0, 128, 1, unroll=4)
            def rbody(r):
                skew_v[c, r, pl.ds(0, 16)] = rows_v[b, r, pl.ds(0, 16)]
                skew_v[c, r, pl.ds(16, 16)] = rows_v[b, r, pl.ds(16, 16)]

        iotas = [lax.iota(jnp.int32, 16) + blk * 16 for blk in range(8)]

        def transpose(c):
            @plsc.parallel_loop(0, _D, 1, unroll=2)
            def tbody(e):
                col = jnp.full((16,), e, jnp.int32)
                for blk in range(8):
                    v = plsc.load_gather(skew_v.at[c], [iotas[blk], col])
                    tile_v[c, e, pl.ds(blk * 16, 16)] = v

        for b in range(_NBUF):
            issue_gather(b, b)

        def body(g, carry):
            for b in range(_NBUF):
                l = _NBUF * g + b
                c = b % 2
                wait_gather(b)
                repack(b, c)

                @pl.when(l + _NBUF < _L)
                def _():
                    issue_gather(l + _NBUF, b)

                @pl.when(l >= 2)
                def _():
                    wait_writes(c)

                transpose(c)
                issue_writes(l, c)
            return carry

        lax.fori_loop(0, _L // _NBUF, body, 0)
        wait_writes(0)
        wait_writes(1)

    return emb


def kernel(x, table):
    idx = x.reshape(_BB, 128, _TR, 8).transpose(2, 0, 3, 1)
    out = _make_emb()(idx, table)
    return out.transpose(2, 4, 0, 1, 3).reshape(_B, _L, _D)


# speedup vs baseline: 1.4695x; 1.0048x over previous
import functools

import jax
import jax.numpy as jnp
from jax import lax
from jax.experimental import pallas as pl
from jax.experimental.pallas import tpu as pltpu
from jax.experimental.pallas import tpu_sc as plsc

_D = 32
_NC = 2
_NS = 16
_NW = _NC * _NS
_B = 4096
_L = 200
_TR = _L // 8
_BB = _B // 128
_EB = _D // 8
_NBUF = 4
_P = _D + 1


@functools.cache
def _make_emb():
    mesh = plsc.VectorSubcoreMesh(core_axis_name="c", subcore_axis_name="s")

    @functools.partial(
        pl.kernel,
        mesh=mesh,
        out_type=jax.ShapeDtypeStruct((_L, _EB, _BB, 8, 128), jnp.float32),
        scratch_types=[
            pltpu.VMEM((_TR, 8, 128), jnp.int32),
            pltpu.VMEM((_NBUF, 128, _D), jnp.float32),
            pltpu.VMEM((2, 128, _P), jnp.float32),
            pltpu.VMEM((2, _D, 128), jnp.float32),
            pltpu.SemaphoreType.DMA((_NBUF,)),
            pltpu.SemaphoreType.DMA((2,)),
        ],
        compiler_params=pltpu.CompilerParams(
            use_tc_tiling_on_sc=False, needs_layout_passes=False),
    )
    def emb(idx_hbm, table_hbm, out_hbm, idx_v, rows_v, skew_v, tile_v,
            gsem, wsem):
        w = lax.axis_index("s") * _NC + lax.axis_index("c")
        pltpu.sync_copy(idx_hbm.at[:, w], idx_v)

        def issue_gather(l, b):
            return pltpu.async_copy(
                table_hbm.at[idx_v.at[l // 8, l % 8]], rows_v.at[b],
                gsem.at[b])

        def wait_gather(b):
            pltpu.make_async_copy(
                table_hbm.at[idx_v.at[0, 0]], rows_v.at[b], gsem.at[b]).wait()

        def issue_writes(l, c):
            for eb in range(_EB):
                pltpu.async_copy(
                    tile_v.at[c, pl.ds(eb * 8, 8)], out_hbm.at[l, eb, w],
                    wsem.at[c])

        def wait_writes(c):
            for eb in range(_EB):
                pltpu.make_async_copy(
                    tile_v.at[c, pl.ds(eb * 8, 8)], out_hbm.at[0, eb, 0],
                    wsem.at[c]).wait()

        def repack(b, c):
            @plsc.parallel_loop(0, 128, 1, unroll=4)
            def rbody(r):
                skew_v[c, r, pl.ds(0, 16)] = rows_v[b, r, pl.ds(0, 16)]
                skew_v[c, r, pl.ds(16, 16)] = rows_v[b, r, pl.ds(16, 16)]

        iotas = [lax.iota(jnp.int32, 16) + blk * 16 for blk in range(8)]

        def transpose(c):
            @plsc.parallel_loop(0, _D, 1, unroll=4)
            def tbody(e):
                col = jnp.full((16,), e, jnp.int32)
                for blk in range(8):
                    v = plsc.load_gather(skew_v.at[c], [iotas[blk], col])
                    tile_v[c, e, pl.ds(blk * 16, 16)] = v

        for b in range(_NBUF):
            issue_gather(b, b)

        def body(g, carry):
            for b in range(_NBUF):
                l = _NBUF * g + b
                c = b % 2
                wait_gather(b)
                repack(b, c)

                @pl.when(l + _NBUF < _L)
                def _():
                    issue_gather(l + _NBUF, b)

                @pl.when(l >= 2)
                def _():
                    wait_writes(c)

                transpose(c)
                issue_writes(l, c)
            return carry

        lax.fori_loop(0, _L // _NBUF, body, 0)
        wait_writes(0)
        wait_writes(1)

    return emb


def kernel(x, table):
    idx = x.reshape(_BB, 128, _TR, 8).transpose(2, 0, 3, 1)
    out = _make_emb()(idx, table)
    return out.transpose(2, 4, 0, 1, 3).reshape(_B, _L, _D)
